# Initial kernel scaffold; baseline (speedup 1.0000x reference)
import jax
import jax.numpy as jnp
from jax.experimental import pallas as pl


def kernel(inputs, mask):
    raise NotImplementedError("write your pallas kernel here")


# speedup vs baseline: 1.6851x; 1.6851x over previous
import jax
import jax.numpy as jnp
from jax.experimental import pallas as pl


def _pool_concat_body(x_ref, m_ref, o_ref):
    ph = pl.program_id(1)

    @pl.when(ph == 0)
    def _():
        o_ref[...] = x_ref[...]

    @pl.when(ph == 1)
    def _():
        x = x_ref[...]
        m = m_ref[...]
        s = jnp.sum(x * m, axis=1)
        cnt = jnp.sum(m)
        mean = s / cnt
        o_ref[...] = jnp.broadcast_to(mean[:, None, :], o_ref.shape)


def kernel(inputs, mask):
    B, S, D = inputs.shape
    mf = mask.astype(inputs.dtype).reshape(B, S, 1)

    out = pl.pallas_call(
        _pool_concat_body,
        grid=(B, 2),
        in_specs=[
            pl.BlockSpec((1, S, D), lambda b, ph: (b, 0, 0)),
            pl.BlockSpec((1, S, 1), lambda b, ph: (b, 0, 0)),
        ],
        out_specs=pl.BlockSpec((1, S, D), lambda b, ph: (b, 0, ph)),
        out_shape=jax.ShapeDtypeStruct((B, S, 2 * D), inputs.dtype),
    )(inputs, mf)
    return out


# speedup vs baseline: 1.8434x; 1.0939x over previous
import jax
import jax.numpy as jnp
from jax.experimental import pallas as pl


def _body(x_ref, m_ref, o_ref):
    x = x_ref[...]
    m = m_ref[...]
    s = jnp.sum(x * m, axis=1)
    cnt = jnp.sum(m)
    mean = s / cnt
    D = x.shape[2]
    o_ref[:, :, :D] = x
    o_ref[:, :, D:] = jnp.broadcast_to(mean[:, None, :], x.shape)


def kernel(inputs, mask):
    B, S, D = inputs.shape
    mf = mask.astype(inputs.dtype).reshape(B, S, 1)

    out = pl.pallas_call(
        _body,
        grid=(B,),
        in_specs=[
            pl.BlockSpec((1, S, D), lambda b: (b, 0, 0)),
            pl.BlockSpec((1, S, 1), lambda b: (b, 0, 0)),
        ],
        out_specs=pl.BlockSpec((1, S, 2 * D), lambda b: (b, 0, 0)),
        out_shape=jax.ShapeDtypeStruct((B, S, 2 * D), inputs.dtype),
    )(inputs, mf)
    return out
